# baseline (speedup 1.0000x reference)
import functools

import jax
import jax.numpy as jnp
from jax import lax
from jax.experimental import pallas as pl
from jax.experimental.pallas import tpu as pltpu
from jax.experimental.pallas import tpu_sc as plsc

NUM_POSES = 100000
POSE_DIM = 9
PAD_DIM = 16
BATCH = 16384

_NUM_CORES = 2
_NUM_SUBCORES = 16
_NUM_WORKERS = _NUM_CORES * _NUM_SUBCORES
_B_PER_W = BATCH // _NUM_WORKERS

_mesh = plsc.VectorSubcoreMesh(core_axis_name="c", subcore_axis_name="s")


@functools.partial(
    pl.kernel,
    mesh=_mesh,
    out_type=jax.ShapeDtypeStruct((BATCH, PAD_DIM), jnp.float32),
    scratch_types=[
        pltpu.VMEM((_B_PER_W,), jnp.int32),
        pltpu.VMEM((_B_PER_W, PAD_DIM), jnp.float32),
        pltpu.SemaphoreType.DMA,
    ],
    compiler_params=pltpu.CompilerParams(use_tc_tiling_on_sc=False),
)
def _gather_sc(tab_hbm, idx_hbm, out_hbm, idx_v, rows_v, sem):
    wid = lax.axis_index("s") * _NUM_CORES + lax.axis_index("c")
    base = wid * _B_PER_W
    pltpu.sync_copy(idx_hbm.at[pl.ds(base, _B_PER_W)], idx_v)
    pltpu.async_copy(tab_hbm.at[idx_v], rows_v, sem).wait()
    pltpu.sync_copy(rows_v, out_hbm.at[pl.ds(base, _B_PER_W)])


def kernel(d9, i):
    tab = jnp.pad(d9, ((0, 0), (0, PAD_DIM - POSE_DIM)))
    out = _gather_sc(tab, i.astype(jnp.int32))
    return out[:, :POSE_DIM]


# speedup vs baseline: 1.1513x; 1.1513x over previous
import functools

import jax
import jax.numpy as jnp
from jax import lax
from jax.experimental import pallas as pl
from jax.experimental.pallas import tpu as pltpu
from jax.experimental.pallas import tpu_sc as plsc

NUM_POSES = 100000
POSE_DIM = 9
BATCH = 16384

_NUM_CORES = 2
_NUM_SUBCORES = 16
_NUM_WORKERS = _NUM_CORES * _NUM_SUBCORES
_B_PER_W = BATCH // _NUM_WORKERS
_W_PER_ROW = POSE_DIM
_OUT_W = _B_PER_W * _W_PER_ROW
_NBLK = NUM_POSES * POSE_DIM // 8
_CHUNK = 128
_LANES = 16

_mesh = plsc.VectorSubcoreMesh(core_axis_name="c", subcore_axis_name="s")


@functools.partial(
    pl.kernel,
    mesh=_mesh,
    out_type=jax.ShapeDtypeStruct((_NUM_WORKERS, _OUT_W), jnp.float32),
    scratch_types=[
        pltpu.VMEM((_B_PER_W,), jnp.int32),
        pltpu.VMEM((_B_PER_W,), jnp.int32),
        pltpu.VMEM((_B_PER_W,), jnp.int32),
        pltpu.VMEM((_B_PER_W,), jnp.int32),
        pltpu.VMEM((2 * _B_PER_W, 8), jnp.float32),
        pltpu.VMEM((_OUT_W,), jnp.float32),
        pltpu.SemaphoreType.DMA,
    ],
    compiler_params=pltpu.CompilerParams(use_tc_tiling_on_sc=False,
                                         needs_layout_passes=False),
)
def _gather_sc(tab_hbm, idx_hbm, out_hbm, idx_v, blka_v, blkb_v, off_v,
               win_v, rows_v, sem):
    wid = lax.axis_index("s") * _NUM_CORES + lax.axis_index("c")
    base = wid * _B_PER_W
    pltpu.sync_copy(idx_hbm.at[pl.ds(base, _B_PER_W)], idx_v)

    def blk_body(t, carry):
        v = idx_v[pl.ds(t * _LANES, _LANES)]
        w = v * _W_PER_ROW
        b = lax.shift_right_logical(w, 3)
        blka_v[pl.ds(t * _LANES, _LANES)] = b
        blkb_v[pl.ds(t * _LANES, _LANES)] = b + 1
        off_v[pl.ds(t * _LANES, _LANES)] = lax.bitwise_and(w, 7)
        return carry

    lax.fori_loop(0, _B_PER_W // _LANES, blk_body, 0)

    copies = []
    for c in range(_B_PER_W // _CHUNK):
        s = c * _CHUNK
        copies.append(pltpu.async_copy(
            tab_hbm.at[blka_v.at[pl.ds(s, _CHUNK)]],
            win_v.at[pl.ds(s, _CHUNK)], sem))
        copies.append(pltpu.async_copy(
            tab_hbm.at[blkb_v.at[pl.ds(s, _CHUNK)]],
            win_v.at[pl.ds(_B_PER_W + s, _CHUNK)], sem))
    for cp in copies:
        cp.wait()

    iota = lax.iota(jnp.int32, _LANES)

    def compact_block(blk_id):
        nbase = blk_id * _LANES
        n = nbase + iota
        k = lax.shift_right_logical(n * 7282, 16)
        j = n - (lax.shift_left(k, 3) + k)
        o = plsc.load_gather(off_v, [k])
        w = o + j
        hi = lax.shift_right_logical(w, 3)
        row = k + lax.shift_left(hi, 9)
        col = w - lax.shift_left(hi, 3)
        rows_v[pl.ds(nbase, _LANES)] = plsc.load_gather(win_v, [row, col])

    def compact_body(t, carry):
        for u in range(4):
            compact_block(t * 4 + u)
        return carry

    lax.fori_loop(0, _OUT_W // _LANES // 4, compact_body, 0)

    pltpu.sync_copy(rows_v, out_hbm.at[wid])


def kernel(d9, i):
    tab = d9.reshape(_NBLK, 8)
    out = _gather_sc(tab, i.astype(jnp.int32))
    return out.reshape(BATCH, POSE_DIM)


# speedup vs baseline: 1.1536x; 1.0021x over previous
import functools

import jax
import jax.numpy as jnp
from jax import lax
from jax.experimental import pallas as pl
from jax.experimental.pallas import tpu as pltpu
from jax.experimental.pallas import tpu_sc as plsc

NUM_POSES = 100000
POSE_DIM = 9
BATCH = 16384

_NUM_CORES = 2
_NUM_SUBCORES = 16
_NUM_WORKERS = _NUM_CORES * _NUM_SUBCORES
_B_PER_W = BATCH // _NUM_WORKERS
_W_PER_ROW = POSE_DIM
_OUT_W = _B_PER_W * _W_PER_ROW
_NBLK = NUM_POSES * POSE_DIM // 8
_CHUNK = 128
_LANES = 16

_mesh = plsc.VectorSubcoreMesh(core_axis_name="c", subcore_axis_name="s")


@functools.partial(
    pl.kernel,
    mesh=_mesh,
    out_type=jax.ShapeDtypeStruct((_NUM_WORKERS, _OUT_W), jnp.float32),
    scratch_types=[
        pltpu.VMEM((_B_PER_W,), jnp.int32),
        pltpu.VMEM((_B_PER_W,), jnp.int32),
        pltpu.VMEM((_B_PER_W,), jnp.int32),
        pltpu.VMEM((_B_PER_W,), jnp.int32),
        pltpu.VMEM((2 * _B_PER_W, 8), jnp.float32),
        pltpu.VMEM((_OUT_W,), jnp.float32),
        pltpu.SemaphoreType.DMA,
    ],
    compiler_params=pltpu.CompilerParams(use_tc_tiling_on_sc=False,
                                         needs_layout_passes=False,
                                         skip_device_barrier=True,
                                         disable_bounds_checks=True,
                                         disable_semaphore_checks=True),
)
def _gather_sc(tab_hbm, idx_hbm, out_hbm, idx_v, blka_v, blkb_v, off_v,
               win_v, rows_v, sem):
    wid = lax.axis_index("s") * _NUM_CORES + lax.axis_index("c")
    base = wid * _B_PER_W
    pltpu.sync_copy(idx_hbm.at[pl.ds(base, _B_PER_W)], idx_v)

    def blk_body(t, carry):
        v = idx_v[pl.ds(t * _LANES, _LANES)]
        w = v * _W_PER_ROW
        b = lax.shift_right_logical(w, 3)
        blka_v[pl.ds(t * _LANES, _LANES)] = b
        blkb_v[pl.ds(t * _LANES, _LANES)] = b + 1
        off_v[pl.ds(t * _LANES, _LANES)] = lax.bitwise_and(w, 7)
        return carry

    lax.fori_loop(0, _B_PER_W // _LANES, blk_body, 0)

    copies = []
    for c in range(_B_PER_W // _CHUNK):
        s = c * _CHUNK
        copies.append(pltpu.async_copy(
            tab_hbm.at[blka_v.at[pl.ds(s, _CHUNK)]],
            win_v.at[pl.ds(s, _CHUNK)], sem))
        copies.append(pltpu.async_copy(
            tab_hbm.at[blkb_v.at[pl.ds(s, _CHUNK)]],
            win_v.at[pl.ds(_B_PER_W + s, _CHUNK)], sem))
    for cp in copies:
        cp.wait()

    iota = lax.iota(jnp.int32, _LANES)

    def compact_block(blk_id):
        nbase = blk_id * _LANES
        n = nbase + iota
        k = lax.shift_right_logical(n * 7282, 16)
        j = n - (lax.shift_left(k, 3) + k)
        o = plsc.load_gather(off_v, [k])
        w = o + j
        hi = lax.shift_right_logical(w, 3)
        row = k + lax.shift_left(hi, 9)
        col = w - lax.shift_left(hi, 3)
        rows_v[pl.ds(nbase, _LANES)] = plsc.load_gather(win_v, [row, col])

    def compact_body(t, carry):
        for u in range(4):
            compact_block(t * 4 + u)
        return carry

    lax.fori_loop(0, _OUT_W // _LANES // 4, compact_body, 0)

    pltpu.sync_copy(rows_v, out_hbm.at[wid])


def kernel(d9, i):
    tab = d9.reshape(_NBLK, 8)
    out = _gather_sc(tab, i.astype(jnp.int32))
    return out.reshape(BATCH, POSE_DIM)


# speedup vs baseline: 3.2317x; 2.8013x over previous
import functools

import jax
import jax.numpy as jnp
from jax import lax
from jax.experimental import pallas as pl
from jax.experimental.pallas import tpu as pltpu
from jax.experimental.pallas import tpu_sc as plsc

NUM_POSES = 100000
POSE_DIM = 9
BATCH = 16384

_NUM_CORES = 2
_NUM_SUBCORES = 16
_NUM_WORKERS = _NUM_CORES * _NUM_SUBCORES
_B_PER_W = BATCH // _NUM_WORKERS
_OUT_W = _B_PER_W * POSE_DIM
_COL_BLKS = NUM_POSES // 8
_LANES = 16

_mesh = plsc.VectorSubcoreMesh(core_axis_name="c", subcore_axis_name="s")


@functools.partial(
    pl.kernel,
    mesh=_mesh,
    out_type=jax.ShapeDtypeStruct((POSE_DIM, BATCH), jnp.float32),
    scratch_types=[
        pltpu.VMEM((_B_PER_W,), jnp.int32),
        pltpu.VMEM((_B_PER_W,), jnp.int32),
        pltpu.VMEM((_OUT_W,), jnp.int32),
        pltpu.VMEM((_OUT_W, 8), jnp.float32),
        pltpu.VMEM((POSE_DIM, _B_PER_W), jnp.float32),
        pltpu.SemaphoreType.DMA,
    ],
    compiler_params=pltpu.CompilerParams(use_tc_tiling_on_sc=False,
                                         needs_layout_passes=False,
                                         skip_device_barrier=True,
                                         disable_bounds_checks=True,
                                         disable_semaphore_checks=True),
)
def _gather_sc(tab_hbm, idx_hbm, out_hbm, idx_v, low_v, il_v, win_v,
               rows_v, sem):
    wid = lax.axis_index("s") * _NUM_CORES + lax.axis_index("c")
    base = wid * _B_PER_W
    pltpu.sync_copy(idx_hbm.at[pl.ds(base, _B_PER_W)], idx_v)

    iota = lax.iota(jnp.int32, _LANES)

    def low_body(t, carry):
        v = idx_v[pl.ds(t * _LANES, _LANES)]
        low_v[pl.ds(t * _LANES, _LANES)] = lax.bitwise_and(v, 7)
        return carry

    lax.fori_loop(0, _B_PER_W // _LANES, low_body, 0)

    def il_block(blk_id):
        n = blk_id * _LANES + iota
        k = lax.shift_right_logical(n * 7282, 16)
        j = n - (lax.shift_left(k, 3) + k)
        r = plsc.load_gather(idx_v, [k])
        il_v[pl.ds(blk_id * _LANES, _LANES)] = (
            j * _COL_BLKS + lax.shift_right_logical(r, 3))

    def il_body(t, carry):
        for u in range(4):
            il_block(t * 4 + u)
        return carry

    lax.fori_loop(0, _OUT_W // _LANES // 4, il_body, 0)

    copies = []
    for c in range(POSE_DIM):
        s = c * _B_PER_W
        copies.append(pltpu.async_copy(
            tab_hbm.at[il_v.at[pl.ds(s, _B_PER_W)]],
            win_v.at[pl.ds(s, _B_PER_W)], sem))
    for cp in copies:
        cp.wait()

    def compact_block(blk_id):
        n = blk_id * _LANES + iota
        k = lax.shift_right_logical(n * 7282, 16)
        j = n - (lax.shift_left(k, 3) + k)
        lo = plsc.load_gather(low_v, [k])
        val = plsc.load_gather(win_v, [n, lo])
        plsc.store_scatter(rows_v, [j, k], val)

    def compact_body(t, carry):
        for u in range(4):
            compact_block(t * 4 + u)
        return carry

    lax.fori_loop(0, _OUT_W // _LANES // 4, compact_body, 0)

    pltpu.sync_copy(rows_v, out_hbm.at[:, pl.ds(base, _B_PER_W)])


def kernel(d9, i):
    tab = d9.T.reshape(NUM_POSES * POSE_DIM // 8, 8)
    out = _gather_sc(tab, i.astype(jnp.int32))
    return out.T


# speedup vs baseline: 3.6593x; 1.1323x over previous
import functools

import jax
import jax.numpy as jnp
from jax import lax
from jax.experimental import pallas as pl
from jax.experimental.pallas import tpu as pltpu
from jax.experimental.pallas import tpu_sc as plsc

NUM_POSES = 100000
POSE_DIM = 9
BATCH = 16384

_NUM_CORES = 2
_NUM_SUBCORES = 16
_NUM_WORKERS = _NUM_CORES * _NUM_SUBCORES
_B_PER_W = BATCH // _NUM_WORKERS
_COL_BLKS = NUM_POSES // 8
_LANES = 16

_mesh = plsc.VectorSubcoreMesh(core_axis_name="c", subcore_axis_name="s")


@functools.partial(
    pl.kernel,
    mesh=_mesh,
    out_type=jax.ShapeDtypeStruct((POSE_DIM, BATCH), jnp.float32),
    scratch_types=[
        pltpu.VMEM((_B_PER_W,), jnp.int32),
        pltpu.VMEM((_B_PER_W,), jnp.int32),
        pltpu.VMEM((POSE_DIM, _B_PER_W), jnp.int32),
        pltpu.VMEM((POSE_DIM * _B_PER_W, 8), jnp.float32),
        pltpu.VMEM((POSE_DIM, _B_PER_W), jnp.float32),
    ] + [pltpu.SemaphoreType.DMA] * POSE_DIM,
    compiler_params=pltpu.CompilerParams(use_tc_tiling_on_sc=False,
                                         needs_layout_passes=False,
                                         skip_device_barrier=True,
                                         disable_bounds_checks=True,
                                         disable_semaphore_checks=True),
)
def _gather_sc(tab_hbm, idx_hbm, out_hbm, idx_v, low_v, il_v, win_v,
               rows_v, *sems):
    wid = lax.axis_index("s") * _NUM_CORES + lax.axis_index("c")
    base = wid * _B_PER_W
    pltpu.sync_copy(idx_hbm.at[pl.ds(base, _B_PER_W)], idx_v)

    iota = lax.iota(jnp.int32, _LANES)

    def prep_body(t, carry):
        v = idx_v[pl.ds(t * _LANES, _LANES)]
        low_v[pl.ds(t * _LANES, _LANES)] = lax.bitwise_and(v, 7)
        b = lax.shift_right_logical(v, 3)
        for j in range(POSE_DIM):
            il_v[j, pl.ds(t * _LANES, _LANES)] = b + j * _COL_BLKS
        return carry

    lax.fori_loop(0, _B_PER_W // _LANES, prep_body, 0)

    copies = [
        pltpu.async_copy(tab_hbm.at[il_v.at[j]],
                         win_v.at[pl.ds(j * _B_PER_W, _B_PER_W)], sems[j])
        for j in range(POSE_DIM)
    ]

    for j in range(POSE_DIM):
        copies[j].wait()

        def col_body(t, carry, j=j):
            lo = low_v[pl.ds(t * _LANES, _LANES)]
            rows16 = (j * _B_PER_W + t * _LANES) + iota
            rows_v[j, pl.ds(t * _LANES, _LANES)] = (
                plsc.load_gather(win_v, [rows16, lo]))
            return carry

        lax.fori_loop(0, _B_PER_W // _LANES, col_body, 0)

    pltpu.sync_copy(rows_v, out_hbm.at[:, pl.ds(base, _B_PER_W)])


def kernel(d9, i):
    tab = d9.T.reshape(NUM_POSES * POSE_DIM // 8, 8)
    out = _gather_sc(tab, i.astype(jnp.int32))
    return out.T
